# Initial kernel scaffold; baseline (speedup 1.0000x reference)
import jax
import jax.numpy as jnp
from jax.experimental import pallas as pl


def kernel(inputs, sparsity_rate):
    raise NotImplementedError("write your pallas kernel here")


# speedup vs baseline: 15.9436x; 15.9436x over previous
import jax
import jax.numpy as jnp
from jax.experimental import pallas as pl
from jax.experimental.pallas import tpu as pltpu

_B = 64
_N = 32768
_RB = 8


def _body(sr_ref, x_ref, o_ref):
    x = x_ref[...]

    sr = jnp.clip(sr_ref[0, 0], 0.0, 1.0)
    idx0 = (sr * jnp.float32(_N)).astype(jnp.int32)
    oob = idx0 >= _N
    idx = jnp.minimum(idx0, _N - 1)
    rank = (_N - 1) - idx

    bits = jax.lax.bitcast_convert_type(x, jnp.int32)
    key = bits ^ ((bits >> 31) & jnp.int32(0x7FFFFFFF))

    lo0 = jnp.full((_RB, 1), jnp.iinfo(jnp.int32).min, jnp.int32)
    hi0 = jnp.full((_RB, 1), jnp.iinfo(jnp.int32).max, jnp.int32)

    def step(_, lohi):
        lo, hi = lohi
        mid = (lo >> 1) + (hi >> 1) + (lo & hi & 1)
        cnt = jnp.sum((key <= mid).astype(jnp.int32), axis=1, keepdims=True)
        take_hi = cnt >= (rank + 1)
        return jnp.where(take_hi, lo, mid + 1), jnp.where(take_hi, mid, hi)

    lo, _ = jax.lax.fori_loop(0, 32, step, (lo0, hi0))
    ti = lo ^ ((lo >> 31) & jnp.int32(0x7FFFFFFF))
    thr = jax.lax.bitcast_convert_type(ti, jnp.float32)
    thr = jnp.where(oob, jnp.float32(jnp.nan), thr)

    mx = jnp.max(x, axis=1, keepdims=True)
    w = jnp.maximum(x + (mx - thr) - mx, 0.0)
    we = w * jnp.exp(x)
    s = jnp.sum(we, axis=1, keepdims=True)
    o_ref[...] = we / s


def kernel(inputs, sparsity_rate):
    sr = sparsity_rate.reshape(1, 1)
    return pl.pallas_call(
        _body,
        grid=(_B // _RB,),
        in_specs=[
            pl.BlockSpec(memory_space=pltpu.SMEM),
            pl.BlockSpec((_RB, _N), lambda i: (i, 0)),
        ],
        out_specs=pl.BlockSpec((_RB, _N), lambda i: (i, 0)),
        out_shape=jax.ShapeDtypeStruct((_B, _N), jnp.float32),
    )(sr, inputs)
